# baseline (speedup 1.0000x reference)
import jax
import jax.numpy as jnp
from jax import lax
from jax.experimental import pallas as pl
from jax.experimental.pallas import tpu as pltpu
from jax.experimental.pallas import tpu_sc as plsc

N_FACTORS = 64
BATCH = 16384
NC = 2
NS = 16
NW = NC * NS
B_PER_W = BATCH // NW
CHUNK = 128
N_CHUNKS = B_PER_W // CHUNK
GPC = CHUNK // 16
CWORDS = N_FACTORS * CHUNK


def _mf_kernel(u_idx_hbm, i_idx_hbm, uf_hbm, if_hbm, ub_hbm, ib_hbm,
               gb_hbm, out_hbm,
               ui_v, ii_v, uw, iw, ubias, ibias, gb_v, out_v,
               sem_a, sem_g):
    wid = lax.axis_index("s") * NC + lax.axis_index("c")

    pltpu.sync_copy(u_idx_hbm.at[wid], ui_v)
    pltpu.sync_copy(i_idx_hbm.at[wid], ii_v)
    pltpu.sync_copy(gb_hbm, gb_v)

    bias_copies = []
    for j in range(N_CHUNKS):
        sl = pl.ds(j * CHUNK, CHUNK)
        bias_copies.append(
            pltpu.async_copy(ub_hbm.at[ui_v.at[sl]], ubias.at[sl], sem_g))
        bias_copies.append(
            pltpu.async_copy(ib_hbm.at[ii_v.at[sl]], ibias.at[sl], sem_g))

    gb = gb_v[...]

    def fire(j, slot):
        usrc = ui_v.at[pl.ds(j * CHUNK, CHUNK)]
        isrc = ii_v.at[pl.ds(j * CHUNK, CHUNK)]
        for c in range(N_FACTORS):
            dst = pl.ds(slot * CWORDS + c * CHUNK, CHUNK)
            pltpu.async_copy(uf_hbm.at[c].at[usrc], uw.at[dst], sem_a)
            pltpu.async_copy(if_hbm.at[c].at[isrc], iw.at[dst], sem_a)

    def drain():
        for c in range(N_FACTORS):
            dst = pl.ds(c * CHUNK, CHUNK)
            pltpu.make_async_copy(ub_hbm.at[pl.ds(0, CHUNK)], uw.at[dst],
                                  sem_a).wait()
            pltpu.make_async_copy(ub_hbm.at[pl.ds(0, CHUNK)], iw.at[dst],
                                  sem_a).wait()

    fire(0, 0)
    fire(1, 1)
    for c in bias_copies:
        c.wait()

    def chunk_body(j, carry):
        slot = lax.rem(j, 2)
        drain()
        for g in range(GPC):
            acc = jnp.zeros((16,), jnp.float32)
            for d in range(N_FACTORS):
                off = slot * CWORDS + d * CHUNK + g * 16
                uv = uw[pl.ds(off, 16)]
                iv = iw[pl.ds(off, 16)]
                acc = acc + uv * iv
            sl = pl.ds(j * CHUNK + g * 16, 16)
            out_v[sl] = acc + ubias[sl] + ibias[sl] + gb

        @pl.when(j + 2 < N_CHUNKS)
        def _():
            fire(j + 2, slot)

        return carry

    lax.fori_loop(0, N_CHUNKS, chunk_body, 0, unroll=False)

    pltpu.sync_copy(out_v, out_hbm.at[pl.ds(wid * B_PER_W, B_PER_W)])


@jax.jit
def kernel(user_idx, item_idx, user_factors, item_factors, user_biases,
           item_biases, global_bias):
    u_idx = user_idx.astype(jnp.int32).reshape(NW, B_PER_W)
    i_idx = item_idx.astype(jnp.int32).reshape(NW, B_PER_W)
    gb16 = jnp.broadcast_to(global_bias.astype(jnp.float32), (16,))
    ub1d = user_biases.reshape(-1)
    ib1d = item_biases.reshape(-1)
    uf_t = user_factors.T
    if_t = item_factors.T

    mesh = plsc.VectorSubcoreMesh(core_axis_name="c", subcore_axis_name="s")
    run = pl.kernel(
        _mf_kernel,
        mesh=mesh,
        out_type=jax.ShapeDtypeStruct((BATCH,), jnp.float32),
        compiler_params=pltpu.CompilerParams(
            needs_layout_passes=False, use_tc_tiling_on_sc=False),
        scratch_types=[
            pltpu.VMEM((B_PER_W,), jnp.int32),
            pltpu.VMEM((B_PER_W,), jnp.int32),
            pltpu.VMEM((2 * CWORDS,), jnp.float32),
            pltpu.VMEM((2 * CWORDS,), jnp.float32),
            pltpu.VMEM((B_PER_W,), jnp.float32),
            pltpu.VMEM((B_PER_W,), jnp.float32),
            pltpu.VMEM((16,), jnp.float32),
            pltpu.VMEM((B_PER_W,), jnp.float32),
            pltpu.SemaphoreType.DMA,
            pltpu.SemaphoreType.DMA,
        ],
    )
    return run(u_idx, i_idx, uf_t, if_t, ub1d, ib1d, gb16)


# speedup vs baseline: 9.5478x; 9.5478x over previous
import jax
import jax.numpy as jnp
from jax import lax
from jax.experimental import pallas as pl
from jax.experimental.pallas import tpu as pltpu
from jax.experimental.pallas import tpu_sc as plsc

N_FACTORS = 64
BATCH = 16384
NC = 2
NS = 16
NW = NC * NS
B_PER_W = BATCH // NW
CHUNK = 128
N_CHUNKS = B_PER_W // CHUNK
GPC = CHUNK // 16
ROWP = 128


def _mf_kernel(u_idx_hbm, i_idx_hbm, uf_hbm, if_hbm, ub_hbm, ib_hbm,
               gb_hbm, out_hbm,
               ui_v, ii_v, uw, iw, ubias, ibias, gb_v, out_v,
               sem_a, sem_g):
    wid = lax.axis_index("s") * NC + lax.axis_index("c")

    pltpu.sync_copy(u_idx_hbm.at[wid], ui_v)
    pltpu.sync_copy(i_idx_hbm.at[wid], ii_v)
    pltpu.sync_copy(gb_hbm, gb_v)

    bias_copies = []
    for j in range(N_CHUNKS):
        sl = pl.ds(j * CHUNK, CHUNK)
        bias_copies.append(
            pltpu.async_copy(ub_hbm.at[ui_v.at[sl]], ubias.at[sl], sem_g))
        bias_copies.append(
            pltpu.async_copy(ib_hbm.at[ii_v.at[sl]], ibias.at[sl], sem_g))

    iota16 = lax.iota(jnp.int32, 16)
    gb = gb_v[...]

    def fire(j, slot):
        src = pl.ds(j * CHUNK, CHUNK)
        dst = pl.ds(slot * CHUNK, CHUNK)
        pltpu.async_copy(uf_hbm.at[ui_v.at[src]], uw.at[dst], sem_a)
        pltpu.async_copy(if_hbm.at[ii_v.at[src]], iw.at[dst], sem_a)

    def drain():
        pltpu.make_async_copy(
            uf_hbm.at[pl.ds(0, CHUNK)], uw.at[pl.ds(0, CHUNK)], sem_a).wait()
        pltpu.make_async_copy(
            if_hbm.at[pl.ds(0, CHUNK)], iw.at[pl.ds(0, CHUNK)], sem_a).wait()

    fire(0, 0)
    fire(1, 1)
    for c in bias_copies:
        c.wait()

    def chunk_body(j, carry):
        slot = lax.rem(j, 2)
        drain()
        for g in range(GPC):
            sl = pl.ds(j * CHUNK + g * 16, 16)
            rows = slot * CHUNK + g * 16 + iota16
            acc = jnp.zeros((16,), jnp.float32)
            for d in range(N_FACTORS):
                dcol = jnp.full((16,), d, jnp.int32)
                uv = plsc.load_gather(uw, [rows, dcol])
                iv = plsc.load_gather(iw, [rows, dcol])
                acc = acc + uv * iv
            out_v[sl] = acc + ubias[sl] + ibias[sl] + gb

        @pl.when(j + 2 < N_CHUNKS)
        def _():
            fire(j + 2, slot)

        return carry

    lax.fori_loop(0, N_CHUNKS, chunk_body, 0, unroll=False)

    pltpu.sync_copy(out_v, out_hbm.at[pl.ds(wid * B_PER_W, B_PER_W)])


@jax.jit
def kernel(user_idx, item_idx, user_factors, item_factors, user_biases,
           item_biases, global_bias):
    u_idx = user_idx.astype(jnp.int32).reshape(NW, B_PER_W)
    i_idx = item_idx.astype(jnp.int32).reshape(NW, B_PER_W)
    gb16 = jnp.broadcast_to(global_bias.astype(jnp.float32), (16,))
    ub1d = user_biases.reshape(-1)
    ib1d = item_biases.reshape(-1)
    ufp = jnp.pad(user_factors, ((0, 0), (0, ROWP - N_FACTORS)))
    ifp = jnp.pad(item_factors, ((0, 0), (0, ROWP - N_FACTORS)))

    mesh = plsc.VectorSubcoreMesh(core_axis_name="c", subcore_axis_name="s")
    run = pl.kernel(
        _mf_kernel,
        mesh=mesh,
        out_type=jax.ShapeDtypeStruct((BATCH,), jnp.float32),
        compiler_params=pltpu.CompilerParams(
            needs_layout_passes=False, use_tc_tiling_on_sc=True),
        scratch_types=[
            pltpu.VMEM((B_PER_W,), jnp.int32),
            pltpu.VMEM((B_PER_W,), jnp.int32),
            pltpu.VMEM((2 * CHUNK, ROWP), jnp.float32),
            pltpu.VMEM((2 * CHUNK, ROWP), jnp.float32),
            pltpu.VMEM((B_PER_W,), jnp.float32),
            pltpu.VMEM((B_PER_W,), jnp.float32),
            pltpu.VMEM((16,), jnp.float32),
            pltpu.VMEM((B_PER_W,), jnp.float32),
            pltpu.SemaphoreType.DMA,
            pltpu.SemaphoreType.DMA,
        ],
    )
    return run(u_idx, i_idx, ufp, ifp, ub1d, ib1d, gb16)
